# baseline (speedup 1.0000x reference)
import jax
import jax.numpy as jnp
from jax.experimental import pallas as pl

_ROWS = 256


def _bcast_body(src_tab_ref, tgt_tab_ref, src_out_ref, tgt_out_ref):
    b = src_out_ref.shape[0]
    src_out_ref[...] = jnp.broadcast_to(src_tab_ref[...][None], (b,) + src_tab_ref.shape)
    tgt_out_ref[...] = jnp.broadcast_to(tgt_tab_ref[...][None], (b,) + tgt_tab_ref.shape)


def kernel(src, tgt, src_pos_table, tgt_pos_table):
    n = src.shape[0]
    src_len = src.shape[1]
    tgt_len = tgt.shape[1]
    embed = src_pos_table.shape[1]

    grid = (src_len // _ROWS,)
    out = pl.pallas_call(
        _bcast_body,
        grid=grid,
        in_specs=[
            pl.BlockSpec((_ROWS, embed), lambda i: (i, 0)),
            pl.BlockSpec((_ROWS, embed), lambda i: (i, 0)),
        ],
        out_specs=[
            pl.BlockSpec((n, _ROWS, embed), lambda i: (0, i, 0)),
            pl.BlockSpec((n, _ROWS, embed), lambda i: (0, i, 0)),
        ],
        out_shape=[
            jax.ShapeDtypeStruct((n, src_len, embed), src_pos_table.dtype),
            jax.ShapeDtypeStruct((n, tgt_len, embed), tgt_pos_table.dtype),
        ],
    )(src_pos_table[:src_len], tgt_pos_table[:tgt_len])
    return (out[0], out[1])


# speedup vs baseline: 1.0325x; 1.0325x over previous
import jax
import jax.numpy as jnp
from jax.experimental import pallas as pl

_ROWS = 512


def _bcast_body(src_tab_ref, tgt_tab_ref, src_out_ref, tgt_out_ref):
    b = src_out_ref.shape[0]
    src_out_ref[...] = jnp.broadcast_to(src_tab_ref[...][None], (b,) + src_tab_ref.shape)
    tgt_out_ref[...] = jnp.broadcast_to(tgt_tab_ref[...][None], (b,) + tgt_tab_ref.shape)


def kernel(src, tgt, src_pos_table, tgt_pos_table):
    n = src.shape[0]
    src_len = src.shape[1]
    tgt_len = tgt.shape[1]
    embed = src_pos_table.shape[1]

    grid = (src_len // _ROWS,)
    out = pl.pallas_call(
        _bcast_body,
        grid=grid,
        in_specs=[
            pl.BlockSpec((_ROWS, embed), lambda i: (i, 0)),
            pl.BlockSpec((_ROWS, embed), lambda i: (i, 0)),
        ],
        out_specs=[
            pl.BlockSpec((n, _ROWS, embed), lambda i: (0, i, 0)),
            pl.BlockSpec((n, _ROWS, embed), lambda i: (0, i, 0)),
        ],
        out_shape=[
            jax.ShapeDtypeStruct((n, src_len, embed), src_pos_table.dtype),
            jax.ShapeDtypeStruct((n, tgt_len, embed), tgt_pos_table.dtype),
        ],
    )(src_pos_table[:src_len], tgt_pos_table[:tgt_len])
    return (out[0], out[1])
